# baseline (speedup 1.0000x reference)
import jax
import jax.numpy as jnp
from jax.experimental import pallas as pl
from jax.experimental.pallas import tpu as pltpu

TOKENS = 2048
D_IN = 1024
N_EXPERTS = 8
D_HID = 1024
D_OUT = 1024
TOP_K = 2
TILE_M = 256


def _moe_kernel(x_ref, gw_ref, gb_ref, w1_ref, b1_ref, w2_ref,
                b2_ref, out_ref, w_scr):
    e = pl.program_id(0)

    @pl.when(e == 0)
    def _gate():
        logits = jnp.dot(x_ref[...], gw_ref[...],
                         preferred_element_type=jnp.float32) + gb_ref[...]
        col = jax.lax.broadcasted_iota(jnp.int32, logits.shape, 1)
        m1 = jnp.max(logits, axis=1, keepdims=True)
        i1 = jnp.min(jnp.where(logits == m1, col, N_EXPERTS), axis=1,
                     keepdims=True)
        l2 = jnp.where(col == i1, -jnp.inf, logits)
        m2 = jnp.max(l2, axis=1, keepdims=True)
        i2 = jnp.min(jnp.where(l2 == m2, col, N_EXPERTS), axis=1,
                     keepdims=True)
        b = jnp.exp(m2 - m1)
        wa = 1.0 / (1.0 + b)
        wb = b / (1.0 + b)
        w_scr[...] = jnp.where(col == i1, wa,
                               jnp.where(col == i2, wb, 0.0))

    w1 = w1_ref[0]
    w2 = w2_ref[0]
    b1v = b1_ref[0]
    b2v = b2_ref[0]

    def body(i, _):
        sl = pl.ds(i * TILE_M, TILE_M)
        h = jnp.maximum(
            jnp.dot(x_ref[sl, :], w1,
                    preferred_element_type=jnp.float32) + b1v, 0.0)
        y = jnp.dot(h, w2, preferred_element_type=jnp.float32) + b2v
        wt = w_scr[sl, :]
        col = jax.lax.broadcasted_iota(jnp.int32, wt.shape, 1)
        we = jnp.sum(jnp.where(col == e, wt, 0.0), axis=1, keepdims=True)
        contrib = we * y

        @pl.when(e == 0)
        def _init():
            out_ref[sl, :] = contrib

        @pl.when(e > 0)
        def _acc():
            out_ref[sl, :] = out_ref[sl, :] + contrib

        return 0

    jax.lax.fori_loop(0, TOKENS // TILE_M, body, 0)


@jax.jit
def kernel(x, gate_W, gate_b, W1, b1, W2, b2):
    gb2d = gate_b.reshape(1, N_EXPERTS)
    b1r = b1.reshape(N_EXPERTS, 1, D_HID)
    b2r = b2.reshape(N_EXPERTS, 1, D_OUT)
    return pl.pallas_call(
        _moe_kernel,
        grid=(N_EXPERTS,),
        in_specs=[
            pl.BlockSpec((TOKENS, D_IN), lambda e: (0, 0)),
            pl.BlockSpec((D_IN, N_EXPERTS), lambda e: (0, 0)),
            pl.BlockSpec((1, N_EXPERTS), lambda e: (0, 0)),
            pl.BlockSpec((1, D_IN, D_HID), lambda e: (e, 0, 0)),
            pl.BlockSpec((1, 1, D_HID), lambda e: (e, 0, 0)),
            pl.BlockSpec((1, D_HID, D_OUT), lambda e: (e, 0, 0)),
            pl.BlockSpec((1, 1, D_OUT), lambda e: (e, 0, 0)),
        ],
        out_specs=pl.BlockSpec((TOKENS, D_OUT), lambda e: (0, 0)),
        out_shape=jax.ShapeDtypeStruct((TOKENS, D_OUT), jnp.float32),
        scratch_shapes=[pltpu.VMEM((TOKENS, N_EXPERTS), jnp.float32)],
    )(x, gate_W, gb2d, W1, b1r, W2, b2r)
